# baseline (speedup 1.0000x reference)
import functools

import jax
import jax.numpy as jnp
from jax import lax
from jax.experimental import pallas as pl
from jax.experimental.pallas import tpu as pltpu
from jax.experimental.pallas import tpu_sc as plsc

N_NODES = 10000
N_EDGES = 320000
BATCH = 10
N_PER = 1000
IN_FEATS = 128
D1 = 64
D2 = 32
HID = 1024
HID2 = 256
OUT_FEATS = 10
EPS = 1e-5

NC = 2
NS = 16
NW = NC * NS
LANES = 16

NP = 10240
STRIPE = NP // NS
CH = 80
EPT = N_EDGES // NW
CPT = EPT // CH
ECH = N_EDGES // CH

_sc_mesh = plsc.VectorSubcoreMesh(core_axis_name="c", subcore_axis_name="s")
_sc_params = pltpu.CompilerParams(use_tc_tiling_on_sc=False)


def _leaky(v):
    return jnp.where(v >= 0, v, 0.01 * v)


DEGW = 16
DFIRE = 8


@functools.partial(
    pl.kernel,
    out_type=jax.ShapeDtypeStruct((NC, NP, DEGW), jnp.float32),
    mesh=_sc_mesh,
    scratch_types=[
        pltpu.VMEM((CPT, CH), jnp.int32),
        pltpu.VMEM((CPT, CH), jnp.int32),
        pltpu.VMEM((CH, DEGW), jnp.float32),
        pltpu.VMEM((CH, DEGW), jnp.float32),
        pltpu.VMEM_SHARED((NP, DEGW), jnp.float32),
        pltpu.SemaphoreType.DMA,
    ],
    compiler_params=_sc_params,
)
def _degree_kernel(src_hbm, dst_hbm, out_hbm, sidx_v, didx_v, ones0_v, ones1_v,
                   deg_sh, sem):
    c = lax.axis_index("c")
    s = lax.axis_index("s")
    wid = s * NC + c

    @pl.loop(0, CH)
    def _(i):
        ones0_v[i, pl.ds(0, DEGW)] = jnp.zeros((DEGW,), jnp.float32)

    @pl.loop(0, STRIPE // CH)
    def _(i):
        pltpu.sync_copy(ones0_v, deg_sh.at[pl.ds(s * STRIPE + i * CH, CH)])

    lane = lax.iota(jnp.int32, LANES)

    @pl.loop(0, CH)
    def _(i):
        ones0_v[i, pl.ds(0, DEGW)] = jnp.where(lane < 8, 1.0, 0.0)
        ones1_v[i, pl.ds(0, DEGW)] = jnp.where(lane < 8, 0.0, 1.0)

    pltpu.sync_copy(src_hbm.at[pl.ds(wid * CPT, CPT)], sidx_v)
    pltpu.sync_copy(dst_hbm.at[pl.ds(wid * CPT, CPT)], didx_v)
    plsc.subcore_barrier()

    @pl.loop(0, CPT // DFIRE)
    def _(jg):
        for b in range(DFIRE):
            j = jg * DFIRE + b
            pltpu.async_copy(ones0_v, deg_sh.at[sidx_v.at[j]], sem, add=True)
            pltpu.async_copy(ones1_v, deg_sh.at[didx_v.at[j]], sem, add=True)
        for b in range(DFIRE):
            j = jg * DFIRE + b
            pltpu.make_async_copy(ones0_v, deg_sh.at[sidx_v.at[j]], sem).wait()
            pltpu.make_async_copy(ones1_v, deg_sh.at[didx_v.at[j]], sem).wait()

    for b in range(CPT % DFIRE):
        j = (CPT // DFIRE) * DFIRE + b
        pltpu.async_copy(ones0_v, deg_sh.at[sidx_v.at[j]], sem, add=True)
        pltpu.async_copy(ones1_v, deg_sh.at[didx_v.at[j]], sem, add=True)
    for b in range(CPT % DFIRE):
        j = (CPT // DFIRE) * DFIRE + b
        pltpu.make_async_copy(ones0_v, deg_sh.at[sidx_v.at[j]], sem).wait()
        pltpu.make_async_copy(ones1_v, deg_sh.at[didx_v.at[j]], sem).wait()

    plsc.subcore_barrier()
    pltpu.sync_copy(deg_sh.at[pl.ds(s * STRIPE, STRIPE)],
                    out_hbm.at[c, pl.ds(s * STRIPE, STRIPE)])


def _make_propagate(D, NBUF, PHASES):
    NGRP = CPT // NBUF
    REM = CPT % NBUF

    @functools.partial(
        pl.kernel,
        out_type=jax.ShapeDtypeStruct((PHASES, NC, NP, D), jnp.float32),
        mesh=_sc_mesh,
        scratch_types=[
            pltpu.VMEM((CPT, CH), jnp.int32),
            pltpu.VMEM((CPT, CH), jnp.int32),
            pltpu.VMEM((CPT, CH), jnp.float32),
            pltpu.VMEM((2, NBUF, CH, D), jnp.float32),
            pltpu.VMEM_SHARED((NP, D), jnp.float32),
            pltpu.SemaphoreType.DMA,
            pltpu.SemaphoreType.DMA,
        ],
        compiler_params=_sc_params,
    )
    def _prop(q_hbm, src_hbm, dst_hbm, w_hbm, out_hbm,
              sidx_v, didx_v, w_v, rows_v, agg_sh, gsem, ssem):
        c = lax.axis_index("c")
        s = lax.axis_index("s")
        wid = s * NC + c

        pltpu.sync_copy(src_hbm.at[pl.ds(wid * CPT, CPT)], sidx_v)
        pltpu.sync_copy(dst_hbm.at[pl.ds(wid * CPT, CPT)], didx_v)
        pltpu.sync_copy(w_hbm.at[pl.ds(wid * CPT, CPT)], w_v)

        @pl.loop(0, PHASES)
        def _(ph):
            @pl.loop(0, CH)
            def _(i):
                for t in range(D // LANES):
                    rows_v[0, 0, i, pl.ds(t * LANES, LANES)] = (
                        jnp.zeros((LANES,), jnp.float32))

            @pl.loop(0, STRIPE // CH)
            def _(i):
                pltpu.sync_copy(rows_v.at[0, 0],
                                agg_sh.at[pl.ds(s * STRIPE + i * CH, CH)])

            plsc.subcore_barrier()

            def gather(j, p, b):
                return pltpu.async_copy(q_hbm.at[ph].at[sidx_v.at[j]],
                                        rows_v.at[p, b], gsem)

            def gather_wait(j, p, b):
                pltpu.make_async_copy(q_hbm.at[ph].at[sidx_v.at[j]],
                                      rows_v.at[p, b], gsem).wait()

            def scale(j, p, b):
                @plsc.parallel_loop(0, CH // LANES, unroll=5)
                def _(eb):
                    wv = w_v[j, pl.ds(eb * LANES, LANES)]
                    for i in range(LANES):
                        ws = wv[i]
                        e = eb * LANES + i
                        for t in range(D // LANES):
                            rows_v[p, b, e, pl.ds(t * LANES, LANES)] = (
                                rows_v[p, b, e, pl.ds(t * LANES, LANES)] * ws)

            def scatter(j, p, b):
                return pltpu.async_copy(rows_v.at[p, b],
                                        agg_sh.at[didx_v.at[j]], ssem,
                                        add=True)

            def scatter_wait(j, p, b):
                pltpu.make_async_copy(rows_v.at[p, b],
                                      agg_sh.at[didx_v.at[j]], ssem).wait()

            for b in range(NBUF):
                gather(b, 0, b)

            @pl.loop(0, NGRP // 2)
            def _(gg):
                for p in range(2):
                    g = gg * 2 + p
                    base = g * NBUF
                    for b in range(NBUF):
                        gather_wait(base + b, p, b)
                        scale(base + b, p, b)
                        scatter(base + b, p, b)
                    @pl.when(g > 0)
                    def _():
                        for b in range(NBUF):
                            scatter_wait((g - 1) * NBUF + b, p ^ 1, b)

                    @pl.when(g + 1 < NGRP)
                    def _():
                        for b in range(NBUF):
                            gather((g + 1) * NBUF + b, p ^ 1, b)

            if NGRP % 2 == 1:
                g = NGRP - 1
                for b in range(NBUF):
                    gather_wait(g * NBUF + b, 0, b)
                    scale(g * NBUF + b, 0, b)
                    scatter(g * NBUF + b, 0, b)
                if NGRP >= 2:
                    for b in range(NBUF):
                        scatter_wait((g - 1) * NBUF + b, 1, b)

            lastp = (NGRP - 1) % 2
            for b in range(NBUF):
                scatter_wait((NGRP - 1) * NBUF + b, lastp, b)

            for r in range(REM):
                j = NGRP * NBUF + r
                gather(j, 0, 0).wait()
                scale(j, 0, 0)
                scatter(j, 0, 0).wait()

            plsc.subcore_barrier()
            pltpu.sync_copy(agg_sh.at[pl.ds(s * STRIPE, STRIPE)],
                            out_hbm.at[ph, c, pl.ds(s * STRIPE, STRIPE)])

    return _prop


_propagate2 = _make_propagate(D1, 5, 2)
_propagate1 = _make_propagate(D1, 5, 1)


def _tc1_body(degp_ref, x_ref, h0_ref, od_ref, id_ref):
    deg = degp_ref[:, 0:2] + degp_ref[:, 2:4]
    dinv = jnp.maximum(deg, 1.0) ** -0.5
    od_ref[...] = dinv[:, 0:1]
    id_ref[...] = dinv[:, 1:2]
    od = dinv[:N_NODES, 0:1]
    h0_ref[0, pl.ds(0, N_NODES), :] = x_ref[:, 0:D1] * od
    h0_ref[1, pl.ds(0, N_NODES), :] = x_ref[:, D1:IN_FEATS] * od


_tc1 = pl.pallas_call(
    _tc1_body,
    out_shape=(
        jax.ShapeDtypeStruct((2, NP, D1), jnp.float32),
        jax.ShapeDtypeStruct((NP, 1), jnp.float32),
        jax.ShapeDtypeStruct((NP, 1), jnp.float32),
    ),
)


def _tc2_body(agg_ref, id_ref, od_ref, w1_ref, g_ref, a_ref, b_ref,
              out_ref):
    alo = (agg_ref[0, 0] + agg_ref[0, 1]) * id_ref[...]
    ahi = (agg_ref[1, 0] + agg_ref[1, 1]) * id_ref[...]
    prod = (jnp.dot(alo, w1_ref[0:D1], preferred_element_type=jnp.float32)
            + jnp.dot(ahi, w1_ref[D1:IN_FEATS],
                      preferred_element_type=jnp.float32))
    h = _leaky(prod)
    mean = jnp.mean(h, axis=0, keepdims=True)
    shift = h - a_ref[...] * mean
    var = jnp.mean(shift * shift, axis=0, keepdims=True)
    hn = g_ref[...] * shift / jnp.sqrt(var + EPS) + b_ref[...]
    out_ref[...] = hn * od_ref[...]


_tc2 = pl.pallas_call(
    _tc2_body,
    grid=(BATCH,),
    in_specs=[
        pl.BlockSpec((2, NC, N_PER, D1), lambda b: (0, 0, b, 0)),
        pl.BlockSpec((N_PER, 1), lambda b: (b, 0)),
        pl.BlockSpec((N_PER, 1), lambda b: (b, 0)),
        pl.BlockSpec((IN_FEATS, D1), lambda b: (0, 0)),
        pl.BlockSpec((1, D1), lambda b: (0, 0)),
        pl.BlockSpec((1, D1), lambda b: (0, 0)),
        pl.BlockSpec((1, D1), lambda b: (0, 0)),
    ],
    out_specs=pl.BlockSpec((N_PER, D1), lambda b: (b, 0)),
    out_shape=jax.ShapeDtypeStruct((NP, D1), jnp.float32),
)


def _tc3a_body(agg_ref, id_ref, w2_ref, g_ref, a_ref, b_ref, out_ref):
    a = (agg_ref[0] + agg_ref[1]) * id_ref[...]
    h = _leaky(jnp.dot(a, w2_ref[...], preferred_element_type=jnp.float32))
    mean = jnp.mean(h, axis=0, keepdims=True)
    shift = h - a_ref[...] * mean
    var = jnp.mean(shift * shift, axis=0, keepdims=True)
    out_ref[0] = g_ref[...] * shift / jnp.sqrt(var + EPS) + b_ref[...]


_tc3a = pl.pallas_call(
    _tc3a_body,
    grid=(BATCH,),
    in_specs=[
        pl.BlockSpec((NC, N_PER, D1), lambda b: (0, b, 0)),
        pl.BlockSpec((N_PER, 1), lambda b: (b, 0)),
        pl.BlockSpec((D1, D2), lambda b: (0, 0)),
        pl.BlockSpec((1, D2), lambda b: (0, 0)),
        pl.BlockSpec((1, D2), lambda b: (0, 0)),
        pl.BlockSpec((1, D2), lambda b: (0, 0)),
    ],
    out_specs=pl.BlockSpec((1, N_PER, D2), lambda b: (b, 0, 0)),
    out_shape=jax.ShapeDtypeStruct((BATCH, N_PER, D2), jnp.float32),
)


KB = 6400
KSTEPS = (N_PER * D2) // KB


def _inst_norm(z):
    m = jnp.mean(z, axis=-1, keepdims=True)
    v = jnp.mean((z - m) ** 2, axis=-1, keepdims=True)
    return (z - m) / jnp.sqrt(v + EPS)


def _tc_mlp_body(flat_ref, win_ref, whid_ref, wcls_ref, out_ref, acc_ref):
    k = pl.program_id(0)

    @pl.when(k == 0)
    def _():
        acc_ref[...] = jnp.zeros_like(acc_ref)

    acc_ref[...] += jnp.dot(flat_ref[...], win_ref[...],
                            preferred_element_type=jnp.float32)

    @pl.when(k == KSTEPS - 1)
    def _():
        z = _inst_norm(_leaky(acc_ref[...]))
        z = _inst_norm(_leaky(jnp.dot(z, whid_ref[...],
                                      preferred_element_type=jnp.float32)))
        out_ref[...] = jnp.dot(z, wcls_ref[...],
                               preferred_element_type=jnp.float32)


_tc_mlp = pl.pallas_call(
    _tc_mlp_body,
    grid=(KSTEPS,),
    in_specs=[
        pl.BlockSpec((BATCH, KB), lambda k: (0, k)),
        pl.BlockSpec((KB, HID), lambda k: (k, 0)),
        pl.BlockSpec((HID, HID2), lambda k: (0, 0)),
        pl.BlockSpec((HID2, OUT_FEATS), lambda k: (0, 0)),
    ],
    out_specs=pl.BlockSpec((BATCH, OUT_FEATS), lambda k: (0, 0)),
    out_shape=jax.ShapeDtypeStruct((BATCH, OUT_FEATS), jnp.float32),
    scratch_shapes=[pltpu.VMEM((BATCH, HID), jnp.float32)],
)


def kernel(x, edge_index, edge_weight, W1, W2, gn1_gamma, gn1_beta, gn1_alpha,
           gn2_gamma, gn2_beta, gn2_alpha, W_in, W_hid, W_cls):
    src = edge_index[0].astype(jnp.int32).reshape(ECH, CH)
    dst = edge_index[1].astype(jnp.int32).reshape(ECH, CH)
    w = edge_weight.astype(jnp.float32).reshape(ECH, CH)

    degp = _degree_kernel(src, dst)
    degp_t = jnp.stack([degp[:, :, 0], degp[:, :, 8]],
                       axis=1).reshape(NC * 2, NP).T
    h0, odinv, idinv = _tc1(degp_t, x)
    agg1 = _propagate2(h0, src, dst, w)
    q2 = _tc2(agg1, idinv, odinv, W1,
              gn1_gamma.reshape(1, D1), gn1_alpha.reshape(1, D1),
              gn1_beta.reshape(1, D1))
    agg2 = _propagate1(q2[None], src, dst, w)
    h2g = _tc3a(agg2.reshape(NC, NP, D1), idinv, W2,
                gn2_gamma.reshape(1, D2), gn2_alpha.reshape(1, D2),
                gn2_beta.reshape(1, D2))
    flat = h2g.reshape(BATCH, N_PER * D2)
    return _tc_mlp(flat, W_in, W_hid, W_cls)


# speedup vs baseline: 1.1314x; 1.1314x over previous
import functools

import jax
import jax.numpy as jnp
from jax import lax
from jax.experimental import pallas as pl
from jax.experimental.pallas import tpu as pltpu
from jax.experimental.pallas import tpu_sc as plsc

N_NODES = 10000
N_EDGES = 320000
BATCH = 10
N_PER = 1000
IN_FEATS = 128
D1 = 64
D2 = 32
HID = 1024
HID2 = 256
OUT_FEATS = 10
EPS = 1e-5

NC = 2
NS = 16
NW = NC * NS
LANES = 16

NP = 10240
STRIPE = NP // NS
CH = 80
EPT = N_EDGES // NW
CPT = EPT // CH
ECH = N_EDGES // CH

_sc_mesh = plsc.VectorSubcoreMesh(core_axis_name="c", subcore_axis_name="s")
_sc_params = pltpu.CompilerParams(use_tc_tiling_on_sc=False)


def _leaky(v):
    return jnp.where(v >= 0, v, 0.01 * v)


DEGW = 16
DFIRE = 8


@functools.partial(
    pl.kernel,
    out_type=jax.ShapeDtypeStruct((NC, NP, DEGW), jnp.float32),
    mesh=_sc_mesh,
    scratch_types=[
        pltpu.VMEM((CPT, CH), jnp.int32),
        pltpu.VMEM((CPT, CH), jnp.int32),
        pltpu.VMEM((CH, DEGW), jnp.float32),
        pltpu.VMEM((CH, DEGW), jnp.float32),
        pltpu.VMEM_SHARED((NP, DEGW), jnp.float32),
        pltpu.SemaphoreType.DMA,
    ],
    compiler_params=_sc_params,
)
def _degree_kernel(src_hbm, dst_hbm, out_hbm, sidx_v, didx_v, ones0_v, ones1_v,
                   deg_sh, sem):
    c = lax.axis_index("c")
    s = lax.axis_index("s")
    wid = s * NC + c

    @pl.loop(0, CH)
    def _(i):
        ones0_v[i, pl.ds(0, DEGW)] = jnp.zeros((DEGW,), jnp.float32)

    @pl.loop(0, STRIPE // CH)
    def _(i):
        pltpu.sync_copy(ones0_v, deg_sh.at[pl.ds(s * STRIPE + i * CH, CH)])

    lane = lax.iota(jnp.int32, LANES)

    @pl.loop(0, CH)
    def _(i):
        ones0_v[i, pl.ds(0, DEGW)] = jnp.where(lane < 8, 1.0, 0.0)
        ones1_v[i, pl.ds(0, DEGW)] = jnp.where(lane < 8, 0.0, 1.0)

    pltpu.sync_copy(src_hbm.at[pl.ds(wid * CPT, CPT)], sidx_v)
    pltpu.sync_copy(dst_hbm.at[pl.ds(wid * CPT, CPT)], didx_v)
    plsc.subcore_barrier()

    @pl.loop(0, CPT // DFIRE)
    def _(jg):
        for b in range(DFIRE):
            j = jg * DFIRE + b
            pltpu.async_copy(ones0_v, deg_sh.at[sidx_v.at[j]], sem, add=True)
            pltpu.async_copy(ones1_v, deg_sh.at[didx_v.at[j]], sem, add=True)
        for b in range(DFIRE):
            j = jg * DFIRE + b
            pltpu.make_async_copy(ones0_v, deg_sh.at[sidx_v.at[j]], sem).wait()
            pltpu.make_async_copy(ones1_v, deg_sh.at[didx_v.at[j]], sem).wait()

    for b in range(CPT % DFIRE):
        j = (CPT // DFIRE) * DFIRE + b
        pltpu.async_copy(ones0_v, deg_sh.at[sidx_v.at[j]], sem, add=True)
        pltpu.async_copy(ones1_v, deg_sh.at[didx_v.at[j]], sem, add=True)
    for b in range(CPT % DFIRE):
        j = (CPT // DFIRE) * DFIRE + b
        pltpu.make_async_copy(ones0_v, deg_sh.at[sidx_v.at[j]], sem).wait()
        pltpu.make_async_copy(ones1_v, deg_sh.at[didx_v.at[j]], sem).wait()

    plsc.subcore_barrier()
    pltpu.sync_copy(deg_sh.at[pl.ds(s * STRIPE, STRIPE)],
                    out_hbm.at[c, pl.ds(s * STRIPE, STRIPE)])


def _make_propagate(D, NBUF, PHASES):
    NGRP = CPT // NBUF
    REM = CPT % NBUF

    @functools.partial(
        pl.kernel,
        out_type=jax.ShapeDtypeStruct((PHASES, NC, NP, D), jnp.float32),
        mesh=_sc_mesh,
        scratch_types=[
            pltpu.VMEM((CPT, CH), jnp.int32),
            pltpu.VMEM((CPT, CH), jnp.int32),
            pltpu.VMEM((CPT, CH), jnp.float32),
            pltpu.VMEM((2, NBUF, CH, D), jnp.float32),
            pltpu.VMEM_SHARED((NP, D), jnp.float32),
            pltpu.SemaphoreType.DMA,
            pltpu.SemaphoreType.DMA,
        ],
        compiler_params=_sc_params,
    )
    def _prop(q_hbm, src_hbm, dst_hbm, w_hbm, out_hbm,
              sidx_v, didx_v, w_v, rows_v, agg_sh, gsem, ssem):
        c = lax.axis_index("c")
        s = lax.axis_index("s")
        wid = s * NC + c

        pltpu.sync_copy(src_hbm.at[pl.ds(wid * CPT, CPT)], sidx_v)
        pltpu.sync_copy(dst_hbm.at[pl.ds(wid * CPT, CPT)], didx_v)
        pltpu.sync_copy(w_hbm.at[pl.ds(wid * CPT, CPT)], w_v)

        @pl.loop(0, PHASES)
        def _(ph):
            @pl.loop(0, CH)
            def _(i):
                for t in range(D // LANES):
                    rows_v[0, 0, i, pl.ds(t * LANES, LANES)] = (
                        jnp.zeros((LANES,), jnp.float32))

            @pl.loop(0, STRIPE // CH)
            def _(i):
                pltpu.sync_copy(rows_v.at[0, 0],
                                agg_sh.at[pl.ds(s * STRIPE + i * CH, CH)])

            plsc.subcore_barrier()

            def gather(j, p, b):
                return pltpu.async_copy(q_hbm.at[ph].at[sidx_v.at[j]],
                                        rows_v.at[p, b], gsem)

            def gather_wait(j, p, b):
                pltpu.make_async_copy(q_hbm.at[ph].at[sidx_v.at[j]],
                                      rows_v.at[p, b], gsem).wait()

            def scale(j, p, b):
                @plsc.parallel_loop(0, CH // LANES)
                def _(eb):
                    wv = w_v[j, pl.ds(eb * LANES, LANES)]
                    for i in range(LANES):
                        ws = wv[i]
                        e = eb * LANES + i
                        for t in range(D // LANES):
                            rows_v[p, b, e, pl.ds(t * LANES, LANES)] = (
                                rows_v[p, b, e, pl.ds(t * LANES, LANES)] * ws)

            def scatter(j, p, b):
                return pltpu.async_copy(rows_v.at[p, b],
                                        agg_sh.at[didx_v.at[j]], ssem,
                                        add=True)

            def scatter_wait(j, p, b):
                pltpu.make_async_copy(rows_v.at[p, b],
                                      agg_sh.at[didx_v.at[j]], ssem).wait()

            for b in range(NBUF):
                gather(b, 0, b)

            @pl.loop(0, NGRP // 2)
            def _(gg):
                for p in range(2):
                    g = gg * 2 + p
                    base = g * NBUF
                    for b in range(NBUF):
                        gather_wait(base + b, p, b)
                        scale(base + b, p, b)
                        scatter(base + b, p, b)
                    @pl.when(g > 0)
                    def _():
                        for b in range(NBUF):
                            scatter_wait((g - 1) * NBUF + b, p ^ 1, b)

                    @pl.when(g + 1 < NGRP)
                    def _():
                        for b in range(NBUF):
                            gather((g + 1) * NBUF + b, p ^ 1, b)

            if NGRP % 2 == 1:
                g = NGRP - 1
                for b in range(NBUF):
                    gather_wait(g * NBUF + b, 0, b)
                    scale(g * NBUF + b, 0, b)
                    scatter(g * NBUF + b, 0, b)
                if NGRP >= 2:
                    for b in range(NBUF):
                        scatter_wait((g - 1) * NBUF + b, 1, b)

            lastp = (NGRP - 1) % 2
            for b in range(NBUF):
                scatter_wait((NGRP - 1) * NBUF + b, lastp, b)

            for r in range(REM):
                j = NGRP * NBUF + r
                gather(j, 0, 0).wait()
                scale(j, 0, 0)
                scatter(j, 0, 0).wait()

            plsc.subcore_barrier()
            pltpu.sync_copy(agg_sh.at[pl.ds(s * STRIPE, STRIPE)],
                            out_hbm.at[ph, c, pl.ds(s * STRIPE, STRIPE)])

    return _prop


_propagate2 = _make_propagate(D1, 5, 2)
_propagate1 = _make_propagate(D1, 5, 1)


def _tc1_body(degp_ref, x_ref, h0_ref, od_ref, id_ref):
    deg = degp_ref[0] + degp_ref[1]
    odinv = jnp.maximum(deg[:, 0:1], 1.0) ** -0.5
    idinv = jnp.maximum(deg[:, 8:9], 1.0) ** -0.5
    od_ref[...] = odinv
    id_ref[...] = idinv
    od = odinv[:N_NODES]
    h0_ref[0, pl.ds(0, N_NODES), :] = x_ref[:, 0:D1] * od
    h0_ref[1, pl.ds(0, N_NODES), :] = x_ref[:, D1:IN_FEATS] * od


_tc1 = pl.pallas_call(
    _tc1_body,
    out_shape=(
        jax.ShapeDtypeStruct((2, NP, D1), jnp.float32),
        jax.ShapeDtypeStruct((NP, 1), jnp.float32),
        jax.ShapeDtypeStruct((NP, 1), jnp.float32),
    ),
)


def _tc2_body(agg_ref, id_ref, od_ref, w1_ref, g_ref, a_ref, b_ref,
              out_ref):
    alo = (agg_ref[0, 0] + agg_ref[0, 1]) * id_ref[...]
    ahi = (agg_ref[1, 0] + agg_ref[1, 1]) * id_ref[...]
    prod = (jnp.dot(alo, w1_ref[0:D1], preferred_element_type=jnp.float32)
            + jnp.dot(ahi, w1_ref[D1:IN_FEATS],
                      preferred_element_type=jnp.float32))
    h = _leaky(prod)
    mean = jnp.mean(h, axis=0, keepdims=True)
    shift = h - a_ref[...] * mean
    var = jnp.mean(shift * shift, axis=0, keepdims=True)
    hn = g_ref[...] * shift / jnp.sqrt(var + EPS) + b_ref[...]
    out_ref[...] = hn * od_ref[...]


_tc2 = pl.pallas_call(
    _tc2_body,
    grid=(BATCH,),
    in_specs=[
        pl.BlockSpec((2, NC, N_PER, D1), lambda b: (0, 0, b, 0)),
        pl.BlockSpec((N_PER, 1), lambda b: (b, 0)),
        pl.BlockSpec((N_PER, 1), lambda b: (b, 0)),
        pl.BlockSpec((IN_FEATS, D1), lambda b: (0, 0)),
        pl.BlockSpec((1, D1), lambda b: (0, 0)),
        pl.BlockSpec((1, D1), lambda b: (0, 0)),
        pl.BlockSpec((1, D1), lambda b: (0, 0)),
    ],
    out_specs=pl.BlockSpec((N_PER, D1), lambda b: (b, 0)),
    out_shape=jax.ShapeDtypeStruct((NP, D1), jnp.float32),
)


def _tc3a_body(agg_ref, id_ref, w2_ref, g_ref, a_ref, b_ref, out_ref):
    a = (agg_ref[0] + agg_ref[1]) * id_ref[...]
    h = _leaky(jnp.dot(a, w2_ref[...], preferred_element_type=jnp.float32))
    mean = jnp.mean(h, axis=0, keepdims=True)
    shift = h - a_ref[...] * mean
    var = jnp.mean(shift * shift, axis=0, keepdims=True)
    out_ref[0] = g_ref[...] * shift / jnp.sqrt(var + EPS) + b_ref[...]


_tc3a = pl.pallas_call(
    _tc3a_body,
    grid=(BATCH,),
    in_specs=[
        pl.BlockSpec((NC, N_PER, D1), lambda b: (0, b, 0)),
        pl.BlockSpec((N_PER, 1), lambda b: (b, 0)),
        pl.BlockSpec((D1, D2), lambda b: (0, 0)),
        pl.BlockSpec((1, D2), lambda b: (0, 0)),
        pl.BlockSpec((1, D2), lambda b: (0, 0)),
        pl.BlockSpec((1, D2), lambda b: (0, 0)),
    ],
    out_specs=pl.BlockSpec((1, N_PER, D2), lambda b: (b, 0, 0)),
    out_shape=jax.ShapeDtypeStruct((BATCH, N_PER, D2), jnp.float32),
)


KB = 6400
KSTEPS = (N_PER * D2) // KB


def _inst_norm(z):
    m = jnp.mean(z, axis=-1, keepdims=True)
    v = jnp.mean((z - m) ** 2, axis=-1, keepdims=True)
    return (z - m) / jnp.sqrt(v + EPS)


def _tc_mlp_body(flat_ref, win_ref, whid_ref, wcls_ref, out_ref, acc_ref):
    k = pl.program_id(0)

    @pl.when(k == 0)
    def _():
        acc_ref[...] = jnp.zeros_like(acc_ref)

    acc_ref[...] += jnp.dot(flat_ref[...], win_ref[...],
                            preferred_element_type=jnp.float32)

    @pl.when(k == KSTEPS - 1)
    def _():
        z = _inst_norm(_leaky(acc_ref[...]))
        z = _inst_norm(_leaky(jnp.dot(z, whid_ref[...],
                                      preferred_element_type=jnp.float32)))
        out_ref[...] = jnp.dot(z, wcls_ref[...],
                               preferred_element_type=jnp.float32)


_tc_mlp = pl.pallas_call(
    _tc_mlp_body,
    grid=(KSTEPS,),
    in_specs=[
        pl.BlockSpec((BATCH, KB), lambda k: (0, k)),
        pl.BlockSpec((KB, HID), lambda k: (k, 0)),
        pl.BlockSpec((HID, HID2), lambda k: (0, 0)),
        pl.BlockSpec((HID2, OUT_FEATS), lambda k: (0, 0)),
    ],
    out_specs=pl.BlockSpec((BATCH, OUT_FEATS), lambda k: (0, 0)),
    out_shape=jax.ShapeDtypeStruct((BATCH, OUT_FEATS), jnp.float32),
    scratch_shapes=[pltpu.VMEM((BATCH, HID), jnp.float32)],
)


def kernel(x, edge_index, edge_weight, W1, W2, gn1_gamma, gn1_beta, gn1_alpha,
           gn2_gamma, gn2_beta, gn2_alpha, W_in, W_hid, W_cls):
    src = edge_index[0].astype(jnp.int32).reshape(ECH, CH)
    dst = edge_index[1].astype(jnp.int32).reshape(ECH, CH)
    w = edge_weight.astype(jnp.float32).reshape(ECH, CH)

    degp = _degree_kernel(src, dst)
    h0, odinv, idinv = _tc1(degp, x)
    agg1 = _propagate2(h0, src, dst, w)
    q2 = _tc2(agg1, idinv, odinv, W1,
              gn1_gamma.reshape(1, D1), gn1_alpha.reshape(1, D1),
              gn1_beta.reshape(1, D1))
    agg2 = _propagate1(q2[None], src, dst, w)
    h2g = _tc3a(agg2.reshape(NC, NP, D1), idinv, W2,
                gn2_gamma.reshape(1, D2), gn2_alpha.reshape(1, D2),
                gn2_beta.reshape(1, D2))
    flat = h2g.reshape(BATCH, N_PER * D2)
    return _tc_mlp(flat, W_in, W_hid, W_cls)


# speedup vs baseline: 1.1321x; 1.0007x over previous
import functools

import jax
import jax.numpy as jnp
from jax import lax
from jax.experimental import pallas as pl
from jax.experimental.pallas import tpu as pltpu
from jax.experimental.pallas import tpu_sc as plsc

N_NODES = 10000
N_EDGES = 320000
BATCH = 10
N_PER = 1000
IN_FEATS = 128
D1 = 64
D2 = 32
HID = 1024
HID2 = 256
OUT_FEATS = 10
EPS = 1e-5

NC = 2
NS = 16
NW = NC * NS
LANES = 16

NP = 10240
STRIPE = NP // NS
CH = 80
EPT = N_EDGES // NW
CPT = EPT // CH
ECH = N_EDGES // CH

_sc_mesh = plsc.VectorSubcoreMesh(core_axis_name="c", subcore_axis_name="s",
                                  num_cores=NC, num_subcores=NS)
_sc_params = pltpu.CompilerParams(use_tc_tiling_on_sc=False)


def _leaky(v):
    return jnp.where(v >= 0, v, 0.01 * v)


DEGW = 16
DFIRE = 8


@functools.partial(
    pl.kernel,
    out_type=jax.ShapeDtypeStruct((NC, NP, DEGW), jnp.float32),
    mesh=_sc_mesh,
    scratch_types=[
        pltpu.VMEM((CPT, CH), jnp.int32),
        pltpu.VMEM((CPT, CH), jnp.int32),
        pltpu.VMEM((CH, DEGW), jnp.float32),
        pltpu.VMEM((CH, DEGW), jnp.float32),
        pltpu.VMEM_SHARED((NP, DEGW), jnp.float32),
        pltpu.SemaphoreType.DMA,
    ],
    compiler_params=_sc_params,
)
def _degree_kernel(src_hbm, dst_hbm, out_hbm, sidx_v, didx_v, ones0_v, ones1_v,
                   deg_sh, sem):
    c = lax.axis_index("c")
    s = lax.axis_index("s")
    wid = s * NC + c

    @pl.loop(0, CH)
    def _(i):
        ones0_v[i, pl.ds(0, DEGW)] = jnp.zeros((DEGW,), jnp.float32)

    @pl.loop(0, STRIPE // CH)
    def _(i):
        pltpu.sync_copy(ones0_v, deg_sh.at[pl.ds(s * STRIPE + i * CH, CH)])

    lane = lax.iota(jnp.int32, LANES)

    @pl.loop(0, CH)
    def _(i):
        ones0_v[i, pl.ds(0, DEGW)] = jnp.where(lane < 8, 1.0, 0.0)
        ones1_v[i, pl.ds(0, DEGW)] = jnp.where(lane < 8, 0.0, 1.0)

    pltpu.sync_copy(src_hbm.at[pl.ds(wid * CPT, CPT)], sidx_v)
    pltpu.sync_copy(dst_hbm.at[pl.ds(wid * CPT, CPT)], didx_v)
    plsc.subcore_barrier()

    @pl.loop(0, CPT // DFIRE)
    def _(jg):
        for b in range(DFIRE):
            j = jg * DFIRE + b
            pltpu.async_copy(ones0_v, deg_sh.at[sidx_v.at[j]], sem, add=True)
            pltpu.async_copy(ones1_v, deg_sh.at[didx_v.at[j]], sem, add=True)
        for b in range(DFIRE):
            j = jg * DFIRE + b
            pltpu.make_async_copy(ones0_v, deg_sh.at[sidx_v.at[j]], sem).wait()
            pltpu.make_async_copy(ones1_v, deg_sh.at[didx_v.at[j]], sem).wait()

    for b in range(CPT % DFIRE):
        j = (CPT // DFIRE) * DFIRE + b
        pltpu.async_copy(ones0_v, deg_sh.at[sidx_v.at[j]], sem, add=True)
        pltpu.async_copy(ones1_v, deg_sh.at[didx_v.at[j]], sem, add=True)
    for b in range(CPT % DFIRE):
        j = (CPT // DFIRE) * DFIRE + b
        pltpu.make_async_copy(ones0_v, deg_sh.at[sidx_v.at[j]], sem).wait()
        pltpu.make_async_copy(ones1_v, deg_sh.at[didx_v.at[j]], sem).wait()

    plsc.subcore_barrier()
    pltpu.sync_copy(deg_sh.at[pl.ds(s * STRIPE, STRIPE)],
                    out_hbm.at[c, pl.ds(s * STRIPE, STRIPE)])


def _make_propagate(D, NBUF, PHASES):
    NGRP = CPT // NBUF
    REM = CPT % NBUF

    @functools.partial(
        pl.kernel,
        out_type=jax.ShapeDtypeStruct((PHASES, NC, NP, D), jnp.float32),
        mesh=_sc_mesh,
        scratch_types=[
            pltpu.VMEM((CPT, CH), jnp.int32),
            pltpu.VMEM((CPT, CH), jnp.int32),
            pltpu.VMEM((CPT, CH), jnp.float32),
            pltpu.VMEM((2, NBUF, CH, D), jnp.float32),
            pltpu.VMEM_SHARED((NP, D), jnp.float32),
            pltpu.SemaphoreType.DMA,
            pltpu.SemaphoreType.DMA,
        ],
        compiler_params=_sc_params,
    )
    def _prop(q_hbm, src_hbm, dst_hbm, w_hbm, out_hbm,
              sidx_v, didx_v, w_v, rows_v, agg_sh, gsem, ssem):
        c = lax.axis_index("c")
        s = lax.axis_index("s")
        wid = s * NC + c

        pltpu.sync_copy(src_hbm.at[pl.ds(wid * CPT, CPT)], sidx_v)
        pltpu.sync_copy(dst_hbm.at[pl.ds(wid * CPT, CPT)], didx_v)
        pltpu.sync_copy(w_hbm.at[pl.ds(wid * CPT, CPT)], w_v)

        @pl.loop(0, PHASES)
        def _(ph):
            @pl.loop(0, CH)
            def _(i):
                for t in range(D // LANES):
                    rows_v[0, 0, i, pl.ds(t * LANES, LANES)] = (
                        jnp.zeros((LANES,), jnp.float32))

            @pl.loop(0, STRIPE // CH)
            def _(i):
                pltpu.sync_copy(rows_v.at[0, 0],
                                agg_sh.at[pl.ds(s * STRIPE + i * CH, CH)])

            plsc.subcore_barrier()

            def gather(j, p, b):
                return pltpu.async_copy(q_hbm.at[ph].at[sidx_v.at[j]],
                                        rows_v.at[p, b], gsem)

            def gather_wait(j, p, b):
                pltpu.make_async_copy(q_hbm.at[ph].at[sidx_v.at[j]],
                                      rows_v.at[p, b], gsem).wait()

            def scale(j, p, b):
                @plsc.parallel_loop(0, CH // LANES)
                def _(eb):
                    wv = w_v[j, pl.ds(eb * LANES, LANES)]
                    for i in range(LANES):
                        ws = wv[i]
                        e = eb * LANES + i
                        for t in range(D // LANES):
                            rows_v[p, b, e, pl.ds(t * LANES, LANES)] = (
                                rows_v[p, b, e, pl.ds(t * LANES, LANES)] * ws)

            def scatter(j, p, b):
                return pltpu.async_copy(rows_v.at[p, b],
                                        agg_sh.at[didx_v.at[j]], ssem,
                                        add=True)

            def scatter_wait(j, p, b):
                pltpu.make_async_copy(rows_v.at[p, b],
                                      agg_sh.at[didx_v.at[j]], ssem).wait()

            for b in range(NBUF):
                gather(b, 0, b)

            @pl.loop(0, NGRP // 2)
            def _(gg):
                for p in range(2):
                    g = gg * 2 + p
                    base = g * NBUF
                    for b in range(NBUF):
                        gather_wait(base + b, p, b)
                        scale(base + b, p, b)
                        scatter(base + b, p, b)
                    @pl.when(g > 0)
                    def _():
                        for b in range(NBUF):
                            scatter_wait((g - 1) * NBUF + b, p ^ 1, b)

                    @pl.when(g + 1 < NGRP)
                    def _():
                        for b in range(NBUF):
                            gather((g + 1) * NBUF + b, p ^ 1, b)

            if NGRP % 2 == 1:
                g = NGRP - 1
                for b in range(NBUF):
                    gather_wait(g * NBUF + b, 0, b)
                    scale(g * NBUF + b, 0, b)
                    scatter(g * NBUF + b, 0, b)
                if NGRP >= 2:
                    for b in range(NBUF):
                        scatter_wait((g - 1) * NBUF + b, 1, b)

            lastp = (NGRP - 1) % 2
            for b in range(NBUF):
                scatter_wait((NGRP - 1) * NBUF + b, lastp, b)

            for r in range(REM):
                j = NGRP * NBUF + r
                gather(j, 0, 0).wait()
                scale(j, 0, 0)
                scatter(j, 0, 0).wait()

            plsc.subcore_barrier()
            pltpu.sync_copy(agg_sh.at[pl.ds(s * STRIPE, STRIPE)],
                            out_hbm.at[ph, c, pl.ds(s * STRIPE, STRIPE)])

    return _prop


_propagate2 = _make_propagate(D1, 5, 2)
_propagate1 = _make_propagate(D1, 5, 1)


def _tc1_body(degp_ref, x_ref, h0_ref, od_ref, id_ref):
    deg = degp_ref[0] + degp_ref[1]
    odinv = jnp.maximum(deg[:, 0:1], 1.0) ** -0.5
    idinv = jnp.maximum(deg[:, 8:9], 1.0) ** -0.5
    od_ref[...] = odinv
    id_ref[...] = idinv
    od = odinv[:N_NODES]
    h0_ref[0, pl.ds(0, N_NODES), :] = x_ref[:, 0:D1] * od
    h0_ref[1, pl.ds(0, N_NODES), :] = x_ref[:, D1:IN_FEATS] * od


_tc1 = pl.pallas_call(
    _tc1_body,
    out_shape=(
        jax.ShapeDtypeStruct((2, NP, D1), jnp.float32),
        jax.ShapeDtypeStruct((NP, 1), jnp.float32),
        jax.ShapeDtypeStruct((NP, 1), jnp.float32),
    ),
)


def _tc2_body(agg_ref, id_ref, od_ref, w1_ref, g_ref, a_ref, b_ref,
              out_ref):
    alo = (agg_ref[0, 0] + agg_ref[0, 1]) * id_ref[...]
    ahi = (agg_ref[1, 0] + agg_ref[1, 1]) * id_ref[...]
    prod = (jnp.dot(alo, w1_ref[0:D1], preferred_element_type=jnp.float32)
            + jnp.dot(ahi, w1_ref[D1:IN_FEATS],
                      preferred_element_type=jnp.float32))
    h = _leaky(prod)
    mean = jnp.mean(h, axis=0, keepdims=True)
    shift = h - a_ref[...] * mean
    var = jnp.mean(shift * shift, axis=0, keepdims=True)
    hn = g_ref[...] * shift / jnp.sqrt(var + EPS) + b_ref[...]
    out_ref[...] = hn * od_ref[...]


_tc2 = pl.pallas_call(
    _tc2_body,
    grid=(BATCH,),
    in_specs=[
        pl.BlockSpec((2, NC, N_PER, D1), lambda b: (0, 0, b, 0)),
        pl.BlockSpec((N_PER, 1), lambda b: (b, 0)),
        pl.BlockSpec((N_PER, 1), lambda b: (b, 0)),
        pl.BlockSpec((IN_FEATS, D1), lambda b: (0, 0)),
        pl.BlockSpec((1, D1), lambda b: (0, 0)),
        pl.BlockSpec((1, D1), lambda b: (0, 0)),
        pl.BlockSpec((1, D1), lambda b: (0, 0)),
    ],
    out_specs=pl.BlockSpec((N_PER, D1), lambda b: (b, 0)),
    out_shape=jax.ShapeDtypeStruct((NP, D1), jnp.float32),
)


def _tc3a_body(agg_ref, id_ref, w2_ref, g_ref, a_ref, b_ref, out_ref):
    a = (agg_ref[0] + agg_ref[1]) * id_ref[...]
    h = _leaky(jnp.dot(a, w2_ref[...], preferred_element_type=jnp.float32))
    mean = jnp.mean(h, axis=0, keepdims=True)
    shift = h - a_ref[...] * mean
    var = jnp.mean(shift * shift, axis=0, keepdims=True)
    out_ref[0] = g_ref[...] * shift / jnp.sqrt(var + EPS) + b_ref[...]


_tc3a = pl.pallas_call(
    _tc3a_body,
    grid=(BATCH,),
    in_specs=[
        pl.BlockSpec((NC, N_PER, D1), lambda b: (0, b, 0)),
        pl.BlockSpec((N_PER, 1), lambda b: (b, 0)),
        pl.BlockSpec((D1, D2), lambda b: (0, 0)),
        pl.BlockSpec((1, D2), lambda b: (0, 0)),
        pl.BlockSpec((1, D2), lambda b: (0, 0)),
        pl.BlockSpec((1, D2), lambda b: (0, 0)),
    ],
    out_specs=pl.BlockSpec((1, N_PER, D2), lambda b: (b, 0, 0)),
    out_shape=jax.ShapeDtypeStruct((BATCH, N_PER, D2), jnp.float32),
)


KB = 6400
KSTEPS = (N_PER * D2) // KB


def _inst_norm(z):
    m = jnp.mean(z, axis=-1, keepdims=True)
    v = jnp.mean((z - m) ** 2, axis=-1, keepdims=True)
    return (z - m) / jnp.sqrt(v + EPS)


def _tc_mlp_body(flat_ref, win_ref, whid_ref, wcls_ref, out_ref, acc_ref):
    k = pl.program_id(0)

    @pl.when(k == 0)
    def _():
        acc_ref[...] = jnp.zeros_like(acc_ref)

    acc_ref[...] += jnp.dot(flat_ref[...], win_ref[...],
                            preferred_element_type=jnp.float32)

    @pl.when(k == KSTEPS - 1)
    def _():
        z = _inst_norm(_leaky(acc_ref[...]))
        z = _inst_norm(_leaky(jnp.dot(z, whid_ref[...],
                                      preferred_element_type=jnp.float32)))
        out_ref[...] = jnp.dot(z, wcls_ref[...],
                               preferred_element_type=jnp.float32)


_tc_mlp = pl.pallas_call(
    _tc_mlp_body,
    grid=(KSTEPS,),
    in_specs=[
        pl.BlockSpec((BATCH, KB), lambda k: (0, k)),
        pl.BlockSpec((KB, HID), lambda k: (k, 0)),
        pl.BlockSpec((HID, HID2), lambda k: (0, 0)),
        pl.BlockSpec((HID2, OUT_FEATS), lambda k: (0, 0)),
    ],
    out_specs=pl.BlockSpec((BATCH, OUT_FEATS), lambda k: (0, 0)),
    out_shape=jax.ShapeDtypeStruct((BATCH, OUT_FEATS), jnp.float32),
    scratch_shapes=[pltpu.VMEM((BATCH, HID), jnp.float32)],
)


def kernel(x, edge_index, edge_weight, W1, W2, gn1_gamma, gn1_beta, gn1_alpha,
           gn2_gamma, gn2_beta, gn2_alpha, W_in, W_hid, W_cls):
    src = edge_index[0].astype(jnp.int32).reshape(ECH, CH)
    dst = edge_index[1].astype(jnp.int32).reshape(ECH, CH)
    w = edge_weight.astype(jnp.float32).reshape(ECH, CH)

    degp = _degree_kernel(src, dst)
    h0, odinv, idinv = _tc1(degp, x)
    agg1 = _propagate2(h0, src, dst, w)
    q2 = _tc2(agg1, idinv, odinv, W1,
              gn1_gamma.reshape(1, D1), gn1_alpha.reshape(1, D1),
              gn1_beta.reshape(1, D1))
    agg2 = _propagate1(q2[None], src, dst, w)
    h2g = _tc3a(agg2.reshape(NC, NP, D1), idinv, W2,
                gn2_gamma.reshape(1, D2), gn2_alpha.reshape(1, D2),
                gn2_beta.reshape(1, D2))
    flat = h2g.reshape(BATCH, N_PER * D2)
    return _tc_mlp(flat, W_in, W_hid, W_cls)
